# baseline (speedup 1.0000x reference)
import jax
import jax.numpy as jnp
from jax import lax
from jax.experimental import pallas as pl
from jax.experimental.pallas import tpu as pltpu
from jax.experimental.pallas import tpu_sc as plsc

_EMB = 64
_BATCH = 4096
_SEQ = 200
_NC, _NS = 2, 16
_NW = _NC * _NS
_ROWS = _BATCH * _SEQ
_RPW = _ROWS // _NW
_CHUNK = 256
_NCHUNK = _RPW // _CHUNK
_LANES = _EMB // 16
_NBUF = 4
_LOOK = 2


def _pos_encoding():
    pos = jnp.arange(_SEQ, dtype=jnp.float32)[:, None]
    i = jnp.arange(_EMB // 2, dtype=jnp.float32)
    div = 10000.0 ** (2.0 * i / _EMB)
    enc = jnp.zeros((_SEQ, _EMB), dtype=jnp.float32)
    enc = enc.at[:, 0::2].set(jnp.sin(pos / div[None, :]))
    enc = enc.at[:, 1::2].set(jnp.cos(pos / div[None, :]))
    return enc


def _sc_body(idx_hbm, table_hbm, enc_hbm, out_hbm, idx_v, rows_v, enc_v,
             *sems):
    gsems = sems[:_NBUF]
    osems = sems[_NBUF:]
    wid = lax.axis_index("s") * _NC + lax.axis_index("c")
    base = wid * _RPW
    pltpu.sync_copy(enc_hbm, enc_v)
    pltpu.sync_copy(idx_hbm.at[pl.ds(base, _RPW)], idx_v)

    def gather_start(c, b):
        for j in range(_CHUNK // 128):
            pltpu.async_copy(
                table_hbm.at[idx_v.at[pl.ds(c * _CHUNK + j * 128, 128)]],
                rows_v.at[b, pl.ds(j * 128, 128)], gsems[b])

    def gather_wait(b):
        pltpu.make_async_copy(
            out_hbm.at[pl.ds(0, _CHUNK)], rows_v.at[b], gsems[b]).wait()

    def out_start(c, b):
        pltpu.async_copy(
            rows_v.at[b], out_hbm.at[pl.ds(base + c * _CHUNK, _CHUNK)],
            osems[b])

    def out_wait(b):
        pltpu.make_async_copy(
            rows_v.at[b], out_hbm.at[pl.ds(base, _CHUNK)], osems[b]).wait()

    for c in range(_LOOK):
        gather_start(c, c)

    def outer(c8, carry):
        for b in range(_NBUF):
            c = c8 + b
            nc = c + _LOOK

            @pl.when(nc < _NCHUNK)
            def _():
                bn = (b + _LOOK) % _NBUF

                @pl.when(c >= _NBUF - _LOOK)
                def _():
                    out_wait(bn)
                gather_start(nc, bn)

            gather_wait(b)
            s0 = lax.rem(c * _CHUNK, _SEQ)

            def add_body(r, acc):
                s = s0 + r
                s = lax.select(s >= _SEQ, s - _SEQ, s)
                s = lax.select(s >= _SEQ, s - _SEQ, s)
                for j in range(_LANES):
                    sl = pl.ds(j * 16, 16)
                    plsc.addupdate(rows_v.at[b, r, sl], enc_v[s, sl])
                return acc

            lax.fori_loop(0, _CHUNK, add_body, 0, unroll=4)
            out_start(c, b)
        return carry

    lax.fori_loop(0, _NCHUNK // _NBUF, lambda i, car: outer(i * _NBUF, car),
                  0)
    for b in range(_NBUF):
        out_wait(b)


def kernel(x, table):
    idx = x.reshape(_ROWS)
    enc = _pos_encoding()
    mesh = plsc.VectorSubcoreMesh(
        core_axis_name="c", subcore_axis_name="s",
        num_cores=_NC, num_subcores=_NS)
    out = pl.kernel(
        _sc_body,
        out_type=jax.ShapeDtypeStruct((_ROWS, _EMB), jnp.float32),
        mesh=mesh,
        scratch_types=[
            pltpu.VMEM((_RPW,), jnp.int32),
            pltpu.VMEM((_NBUF, _CHUNK, _EMB), jnp.float32),
            pltpu.VMEM((_SEQ, _EMB), jnp.float32),
        ] + [pltpu.SemaphoreType.DMA] * (2 * _NBUF),
        compiler_params=pltpu.CompilerParams(use_tc_tiling_on_sc=False),
    )(idx, table, enc)
    return out.reshape(_BATCH, _SEQ, _EMB)


# speedup vs baseline: 1.1743x; 1.1743x over previous
import jax
import jax.numpy as jnp
from jax import lax
from jax.experimental import pallas as pl
from jax.experimental.pallas import tpu as pltpu
from jax.experimental.pallas import tpu_sc as plsc

_EMB = 64
_BATCH = 4096
_SEQ = 200
_NC, _NS = 2, 16
_NW = _NC * _NS
_ROWS = _BATCH * _SEQ
_RPW = _ROWS // _NW
_CHUNK = 200
_NCHUNK = _RPW // _CHUNK
_LANES = _EMB // 16
_NBUF = 4
_LOOK = 2


def _pos_encoding():
    pos = jnp.arange(_SEQ, dtype=jnp.float32)[:, None]
    i = jnp.arange(_EMB // 2, dtype=jnp.float32)
    div = 10000.0 ** (2.0 * i / _EMB)
    enc = jnp.zeros((_SEQ, _EMB), dtype=jnp.float32)
    enc = enc.at[:, 0::2].set(jnp.sin(pos / div[None, :]))
    enc = enc.at[:, 1::2].set(jnp.cos(pos / div[None, :]))
    return enc


def _sc_body(idx_hbm, table_hbm, enc_hbm, out_hbm, idx_v, rows_v, enc_v,
             *sems):
    gsems = sems[:_NBUF]
    osems = sems[_NBUF:]
    wid = lax.axis_index("s") * _NC + lax.axis_index("c")
    base = wid * _RPW
    pltpu.sync_copy(enc_hbm, enc_v)
    pltpu.sync_copy(idx_hbm.at[pl.ds(base, _RPW)], idx_v)

    def gather_start(c, b):
        off = 0
        while off < _CHUNK:
            n = min(128, _CHUNK - off)
            pltpu.async_copy(
                table_hbm.at[idx_v.at[pl.ds(c * _CHUNK + off, n)]],
                rows_v.at[b, pl.ds(off, n)], gsems[b])
            off += n

    def gather_wait(b):
        pltpu.make_async_copy(
            out_hbm.at[pl.ds(0, _CHUNK)], rows_v.at[b], gsems[b]).wait()

    def out_start(c, b):
        pltpu.async_copy(
            rows_v.at[b], out_hbm.at[pl.ds(base + c * _CHUNK, _CHUNK)],
            osems[b])

    def out_wait(b):
        pltpu.make_async_copy(
            rows_v.at[b], out_hbm.at[pl.ds(base, _CHUNK)], osems[b]).wait()

    for c in range(_LOOK):
        gather_start(c, c)

    def outer(c8, carry):
        for b in range(_NBUF):
            c = c8 + b
            nc = c + _LOOK

            @pl.when(nc < _NCHUNK)
            def _():
                bn = (b + _LOOK) % _NBUF

                @pl.when(c >= _NBUF - _LOOK)
                def _():
                    out_wait(bn)
                gather_start(nc, bn)

            gather_wait(b)
            s0 = lax.rem(c * _CHUNK, _SEQ)

            def add_body(r, acc):
                s = s0 + r
                s = lax.select(s >= _SEQ, s - _SEQ, s)
                s = lax.select(s >= _SEQ, s - _SEQ, s)
                for j in range(_LANES):
                    sl = pl.ds(j * 16, 16)
                    plsc.addupdate(rows_v.at[b, r, sl], enc_v[s, sl])
                return acc

            lax.fori_loop(0, _CHUNK, add_body, 0, unroll=4)
            out_start(c, b)
        return carry

    lax.fori_loop(0, _NCHUNK // _NBUF, lambda i, car: outer(i * _NBUF, car),
                  0)
    for b in range(_NBUF):
        out_wait(b)


def kernel(x, table):
    idx = x.reshape(_ROWS)
    enc = _pos_encoding()
    mesh = plsc.VectorSubcoreMesh(
        core_axis_name="c", subcore_axis_name="s",
        num_cores=_NC, num_subcores=_NS)
    out = pl.kernel(
        _sc_body,
        out_type=jax.ShapeDtypeStruct((_ROWS, _EMB), jnp.float32),
        mesh=mesh,
        scratch_types=[
            pltpu.VMEM((_RPW,), jnp.int32),
            pltpu.VMEM((_NBUF, _CHUNK, _EMB), jnp.float32),
            pltpu.VMEM((_SEQ, _EMB), jnp.float32),
        ] + [pltpu.SemaphoreType.DMA] * (2 * _NBUF),
        compiler_params=pltpu.CompilerParams(use_tc_tiling_on_sc=False),
    )(idx, table, enc)
    return out.reshape(_BATCH, _SEQ, _EMB)


# speedup vs baseline: 1.5567x; 1.3256x over previous
import jax
import jax.numpy as jnp
from jax import lax
from jax.experimental import pallas as pl
from jax.experimental.pallas import tpu as pltpu
from jax.experimental.pallas import tpu_sc as plsc

_EMB = 64
_BATCH = 4096
_SEQ = 200
_NC, _NS = 2, 16
_NW = _NC * _NS
_ROWS = _BATCH * _SEQ
_RPW = _ROWS // _NW
_CHUNK = 200
_NCHUNK = _RPW // _CHUNK
_LANES = _EMB // 16
_NBUF = 4
_LOOK = 2


def _pos_encoding():
    pos = jnp.arange(_SEQ, dtype=jnp.float32)[:, None]
    i = jnp.arange(_EMB // 2, dtype=jnp.float32)
    div = 10000.0 ** (2.0 * i / _EMB)
    enc = jnp.zeros((_SEQ, _EMB), dtype=jnp.float32)
    enc = enc.at[:, 0::2].set(jnp.sin(pos / div[None, :]))
    enc = enc.at[:, 1::2].set(jnp.cos(pos / div[None, :]))
    return enc


def _sc_body(idx_hbm, table_hbm, enc_hbm, out_hbm, idx_v, rows_v, enc_v,
             *sems):
    gsems = sems[:_NBUF]
    osems = sems[_NBUF:]
    wid = lax.axis_index("s") * _NC + lax.axis_index("c")
    base = wid * _RPW
    pltpu.sync_copy(enc_hbm, enc_v)
    pltpu.sync_copy(idx_hbm.at[pl.ds(base, _RPW)], idx_v)

    def gather_start(c, b):
        off = 0
        while off < _CHUNK:
            n = min(128, _CHUNK - off)
            pltpu.async_copy(
                table_hbm.at[idx_v.at[pl.ds(c * _CHUNK + off, n)]],
                rows_v.at[b, pl.ds(off, n)], gsems[b])
            off += n

    def gather_wait(b):
        pltpu.make_async_copy(
            out_hbm.at[pl.ds(0, _CHUNK), pl.ds(0, _EMB)],
            rows_v.at[b], gsems[b]).wait()

    def out_start(c, b):
        pltpu.async_copy(
            rows_v.at[b],
            out_hbm.at[pl.ds(base + c * _CHUNK, _CHUNK), pl.ds(0, _EMB)],
            osems[b])

    def out_wait(b):
        pltpu.make_async_copy(
            rows_v.at[b],
            out_hbm.at[pl.ds(base, _CHUNK), pl.ds(0, _EMB)],
            osems[b]).wait()

    for c in range(_LOOK):
        gather_start(c, c)

    def outer(c8, carry):
        for b in range(_NBUF):
            c = c8 + b
            nc = c + _LOOK

            @pl.when(nc < _NCHUNK)
            def _():
                bn = (b + _LOOK) % _NBUF

                @pl.when(c >= _NBUF - _LOOK)
                def _():
                    out_wait(bn)
                gather_start(nc, bn)

            gather_wait(b)
            s0 = lax.rem(c * _CHUNK, _SEQ)

            def add_body(r, acc):
                s = s0 + r
                s = lax.select(s >= _SEQ, s - _SEQ, s)
                s = lax.select(s >= _SEQ, s - _SEQ, s)
                for j in range(_LANES):
                    sl = pl.ds(j * 16, 16)
                    plsc.addupdate(rows_v.at[b, r, sl], enc_v[s, sl])
                return acc

            lax.fori_loop(0, _CHUNK, add_body, 0, unroll=4)
            out_start(c, b)
        return carry

    lax.fori_loop(0, _NCHUNK // _NBUF, lambda i, car: outer(i * _NBUF, car),
                  0)
    for b in range(_NBUF):
        out_wait(b)


def kernel(x, table):
    idx = x.reshape(_ROWS)
    enc = _pos_encoding()
    mesh = plsc.VectorSubcoreMesh(
        core_axis_name="c", subcore_axis_name="s",
        num_cores=_NC, num_subcores=_NS)
    out = pl.kernel(
        _sc_body,
        out_type=jax.ShapeDtypeStruct((_ROWS, 128), jnp.float32),
        mesh=mesh,
        scratch_types=[
            pltpu.VMEM((_RPW,), jnp.int32),
            pltpu.VMEM((_NBUF, _CHUNK, _EMB), jnp.float32),
            pltpu.VMEM((_SEQ, _EMB), jnp.float32),
        ] + [pltpu.SemaphoreType.DMA] * (2 * _NBUF),
        compiler_params=pltpu.CompilerParams(use_tc_tiling_on_sc=False),
    )(idx, table, enc)
    return out[:, :_EMB].reshape(_BATCH, _SEQ, _EMB)
